# baseline (speedup 1.0000x reference)
import functools

import jax
import jax.numpy as jnp
from jax import lax
from jax.experimental import pallas as pl
from jax.experimental.pallas import tpu as pltpu
from jax.experimental.pallas import tpu_sc as plsc

N = 10000
E = 640000
D_IN = 128
D_H = 64

NC = 2
NS = 16
NW = NC * NS
B = 80
CPW = E // (NW * B)
NPAD = 10240
RPS = NPAD // NS
CNTW = 16

_f32 = jnp.float32

_MESH = plsc.VectorSubcoreMesh(core_axis_name="c", subcore_axis_name="s")
_SC_PARAMS = pltpu.CompilerParams(use_tc_tiling_on_sc=False)


def _zero_rows(ref, rows, width):
    zero16 = jnp.zeros((16,), _f32)

    def zrow(r, _):
        for k in range(width // 16):
            ref[r, pl.ds(k * 16, 16)] = zero16
        return 0

    lax.fori_loop(0, rows, zrow, 0)


NBUF = 5
GROUPS = CPW // NBUF


def _make_seg_sum(with_counts):
    def body(rows_hbm, src_hbm, dst_hbm, *rest):
        if with_counts:
            (out_sum, out_cnt, acc_sh, cnt_sh, src_v, dst_v, gath_v, zbuf_v,
             ones_v, zcnt_v, csem, *sems) = rest
        else:
            out_sum, acc_sh, src_v, dst_v, gath_v, zbuf_v, *sems = rest
        gsem = sems[:NBUF]
        ssem = sems[NBUF:]

        c = lax.axis_index("c")
        s = lax.axis_index("s")
        wid = s * NC + c

        _zero_rows(zbuf_v, RPS // 4, D_H)
        for q in range(4):
            pltpu.sync_copy(zbuf_v, acc_sh.at[pl.ds(s * RPS + q * (RPS // 4),
                                                    RPS // 4)])
        if with_counts:
            _zero_rows(zcnt_v, RPS // 4, CNTW)
            one16 = jnp.ones((16,), _f32)

            def orow(r, _):
                ones_v[r, pl.ds(0, CNTW)] = one16
                return 0

            lax.fori_loop(0, B, orow, 0)
            for q in range(4):
                pltpu.sync_copy(zcnt_v,
                                cnt_sh.at[pl.ds(s * RPS + q * (RPS // 4),
                                                RPS // 4)])

        pltpu.sync_copy(src_hbm.at[wid], src_v)
        pltpu.sync_copy(dst_hbm.at[wid], dst_v)

        plsc.subcore_barrier()

        def gather(chunk, b):
            pltpu.async_copy(rows_hbm.at[src_v.at[chunk]], gath_v.at[b],
                             gsem[b])

        def gwait(b):
            pltpu.make_async_copy(rows_hbm.at[src_v.at[0]],
                                  gath_v.at[b], gsem[b]).wait()

        def scatter(chunk, b):
            pltpu.async_copy(gath_v.at[b], acc_sh.at[dst_v.at[chunk]],
                             ssem[b], add=True)

        def swait(b):
            pltpu.make_async_copy(gath_v.at[b], acc_sh.at[dst_v.at[0]],
                                  ssem[b]).wait()

        def cscatter(chunk):
            pltpu.async_copy(ones_v, cnt_sh.at[dst_v.at[chunk]], csem,
                             add=True)

        def cwait():
            pltpu.make_async_copy(ones_v, cnt_sh.at[dst_v.at[0]],
                                  csem).wait()

        for b in range(NBUF):
            gather(b, b)

        def group(g, _):
            for b in range(NBUF):
                chunk = g * NBUF + b
                gwait(b)
                scatter(chunk, b)
                if with_counts:
                    @pl.when(chunk > 0)
                    def _():
                        cwait()
                    cscatter(chunk)
            for b in range(NBUF):
                chunk = g * NBUF + b
                swait(b)
                gather(chunk + NBUF, b)
            return 0

        lax.fori_loop(0, GROUPS - 1, group, 0)

        for b in range(NBUF):
            chunk = (GROUPS - 1) * NBUF + b
            gwait(b)
            scatter(chunk, b)
            if with_counts:
                cwait()
                cscatter(chunk)
        for b in range(NBUF):
            swait(b)
        if with_counts:
            cwait()

        plsc.subcore_barrier()

        for q in range(4):
            base = s * RPS + q * (RPS // 4)
            pltpu.sync_copy(acc_sh.at[pl.ds(base, RPS // 4)], zbuf_v)
            pltpu.sync_copy(zbuf_v, out_sum.at[c, pl.ds(base, RPS // 4)])
        if with_counts:
            for q in range(4):
                base = s * RPS + q * (RPS // 4)
                pltpu.sync_copy(cnt_sh.at[pl.ds(base, RPS // 4)], zcnt_v)
                pltpu.sync_copy(zcnt_v, out_cnt.at[c, pl.ds(base, RPS // 4)])

    out_type = [pltpu.HBM((NC, NPAD, D_H), _f32)]
    scratch = [
        pltpu.VMEM_SHARED((NPAD, D_H), _f32),
    ]
    if with_counts:
        out_type.append(pltpu.HBM((NC, NPAD, CNTW), _f32))
        scratch.append(pltpu.VMEM_SHARED((NPAD, CNTW), _f32))
    scratch += [
        pltpu.VMEM((CPW, B), jnp.int32),
        pltpu.VMEM((CPW, B), jnp.int32),
        pltpu.VMEM((NBUF, B, D_H), _f32),
        pltpu.VMEM((RPS // 4, D_H), _f32),
    ]
    if with_counts:
        scratch += [
            pltpu.VMEM((B, CNTW), _f32),
            pltpu.VMEM((RPS // 4, CNTW), _f32),
            pltpu.SemaphoreType.DMA,
        ]
    scratch += [pltpu.SemaphoreType.DMA] * (2 * NBUF)

    return pl.kernel(body, mesh=_MESH, out_type=out_type,
                     scratch_types=scratch, compiler_params=_SC_PARAMS)


_seg_sum_counts = _make_seg_sum(True)
_seg_sum_plain = _make_seg_sum(False)


def _dot_t(a, w):
    return lax.dot_general(a, w, (((1,), (1,)), ((), ())),
                           preferred_element_type=_f32)


def _dense_in_body(x_ref, wl_ref, wr_ref, b_ref, xl_ref, sf_ref):
    x = x_ref[...]
    xl_ref[...] = _dot_t(x, wl_ref[...])
    sf_ref[...] = _dot_t(x, wr_ref[...]) + b_ref[...]


_dense_in = pl.pallas_call(
    _dense_in_body,
    out_shape=(jax.ShapeDtypeStruct((N, D_H), _f32),
               jax.ShapeDtypeStruct((N, D_H), _f32)),
)


def _mid_body(p_ref, c_ref, sf_ref, wl_ref, wr_ref, b_ref, hl_ref, sf2_ref):
    ssum = p_ref[0, :N, :] + p_ref[1, :N, :]
    cnt = c_ref[0, :N, 0:1] + c_ref[1, :N, 0:1]
    h = jnp.maximum(ssum / jnp.maximum(cnt, 1.0) + sf_ref[...], 0.0)
    hl_ref[...] = _dot_t(h, wl_ref[...])
    sf2_ref[...] = _dot_t(h, wr_ref[...]) + b_ref[...]


_mid = pl.pallas_call(
    _mid_body,
    out_shape=(jax.ShapeDtypeStruct((N, D_H), _f32),
               jax.ShapeDtypeStruct((N, D_H), _f32)),
)


def _final_body(p_ref, c_ref, sf_ref, wo_ref, bo_ref, out_ref):
    ssum = p_ref[0, :N, :] + p_ref[1, :N, :]
    cnt = c_ref[0, :N, 0:1] + c_ref[1, :N, 0:1]
    h = jnp.maximum(ssum / jnp.maximum(cnt, 1.0) + sf_ref[...], 0.0)
    out_ref[...] = _dot_t(h, wo_ref[...]) + bo_ref[...]


_final = pl.pallas_call(
    _final_body,
    out_shape=jax.ShapeDtypeStruct((N, 128), _f32),
)


def kernel(x, edge_index, W1l, b1l, W1r, W2l, b2l, W2r, Wout, bout):
    src3 = edge_index[0].reshape(NW, CPW, B)
    dst3 = edge_index[1].reshape(NW, CPW, B)

    xl, sf1 = _dense_in(x, W1l, W1r, b1l.reshape(1, D_H))
    psum1, pcnt = _seg_sum_counts(xl, src3, dst3)
    hl, sf2 = _mid(psum1, pcnt, sf1, W2l, W2r, b2l.reshape(1, D_H))
    psum2, = _seg_sum_plain(hl, src3, dst3)

    wo_pad = jnp.zeros((128, D_H), _f32).at[:2, :].set(Wout)
    bo_pad = jnp.zeros((1, 128), _f32).at[0, :2].set(bout)
    out_pad = _final(psum2, pcnt, sf2, wo_pad, bo_pad)
    return out_pad[:, :2]


# speedup vs baseline: 1.1449x; 1.1449x over previous
import functools

import jax
import jax.numpy as jnp
from jax import lax
from jax.experimental import pallas as pl
from jax.experimental.pallas import tpu as pltpu
from jax.experimental.pallas import tpu_sc as plsc

N = 10000
E = 640000
D_IN = 128
D_H = 64

NC = 2
NS = 16
NW = NC * NS
B = 80
CPW = E // (NW * B)
NPAD = 10240
RPS = NPAD // NS
CNTW = 16

_f32 = jnp.float32

_MESH = plsc.VectorSubcoreMesh(core_axis_name="c", subcore_axis_name="s")
_SC_PARAMS = pltpu.CompilerParams(use_tc_tiling_on_sc=False)


def _zero_rows(ref, rows, width):
    zero16 = jnp.zeros((16,), _f32)

    def zrow(r, _):
        for k in range(width // 16):
            ref[r, pl.ds(k * 16, 16)] = zero16
        return 0

    lax.fori_loop(0, rows, zrow, 0)


NBUF = 5
GROUPS = CPW // NBUF


def _make_seg_sum(with_counts):
    def body(rows_hbm, src_hbm, dst_hbm, *rest):
        if with_counts:
            (out_sum, out_cnt, acc_sh, cnt_sh, src_v, dst_v, gath_v, zbuf_v,
             ones_v, zcnt_v, csem, *gsem) = rest
        else:
            out_sum, acc_sh, src_v, dst_v, gath_v, zbuf_v, *gsem = rest

        c = lax.axis_index("c")
        s = lax.axis_index("s")
        wid = s * NC + c

        _zero_rows(zbuf_v, RPS // 4, D_H)
        for q in range(4):
            pltpu.sync_copy(zbuf_v, acc_sh.at[pl.ds(s * RPS + q * (RPS // 4),
                                                    RPS // 4)])
        if with_counts:
            _zero_rows(zcnt_v, RPS // 4, CNTW)
            one16 = jnp.ones((16,), _f32)

            def orow(r, _):
                ones_v[r, pl.ds(0, CNTW)] = one16
                return 0

            lax.fori_loop(0, B, orow, 0)
            for q in range(4):
                pltpu.sync_copy(zcnt_v,
                                cnt_sh.at[pl.ds(s * RPS + q * (RPS // 4),
                                                RPS // 4)])

        pltpu.sync_copy(src_hbm.at[wid], src_v)
        pltpu.sync_copy(dst_hbm.at[wid], dst_v)

        plsc.subcore_barrier()

        def gather(chunk, b):
            pltpu.async_copy(rows_hbm.at[src_v.at[chunk]], gath_v.at[b],
                             gsem[b])

        def gwait(b):
            pltpu.make_async_copy(rows_hbm.at[src_v.at[0]],
                                  gath_v.at[b], gsem[b]).wait()

        def scatter(chunk, b):
            pltpu.sync_copy(gath_v.at[b], acc_sh.at[dst_v.at[chunk]],
                            add=True)

        def cscatter(chunk):
            pltpu.async_copy(ones_v, cnt_sh.at[dst_v.at[chunk]], csem,
                             add=True)

        def cwait():
            pltpu.make_async_copy(ones_v, cnt_sh.at[dst_v.at[0]],
                                  csem).wait()

        for b in range(NBUF):
            gather(b, b)

        def group(g, _):
            for b in range(NBUF):
                chunk = g * NBUF + b
                gwait(b)
                if with_counts:
                    @pl.when(chunk > 0)
                    def _():
                        cwait()
                    cscatter(chunk)
                scatter(chunk, b)
                gather(chunk + NBUF, b)
            return 0

        lax.fori_loop(0, GROUPS - 1, group, 0)

        for b in range(NBUF):
            chunk = (GROUPS - 1) * NBUF + b
            gwait(b)
            if with_counts:
                cwait()
                cscatter(chunk)
            scatter(chunk, b)
        if with_counts:
            cwait()

        plsc.subcore_barrier()

        for q in range(4):
            base = s * RPS + q * (RPS // 4)
            pltpu.sync_copy(acc_sh.at[pl.ds(base, RPS // 4)], zbuf_v)
            pltpu.sync_copy(zbuf_v, out_sum.at[c, pl.ds(base, RPS // 4)])
        if with_counts:
            for q in range(4):
                base = s * RPS + q * (RPS // 4)
                pltpu.sync_copy(cnt_sh.at[pl.ds(base, RPS // 4)], zcnt_v)
                pltpu.sync_copy(zcnt_v, out_cnt.at[c, pl.ds(base, RPS // 4)])

    out_type = [pltpu.HBM((NC, NPAD, D_H), _f32)]
    scratch = [
        pltpu.VMEM_SHARED((NPAD, D_H), _f32),
    ]
    if with_counts:
        out_type.append(pltpu.HBM((NC, NPAD, CNTW), _f32))
        scratch.append(pltpu.VMEM_SHARED((NPAD, CNTW), _f32))
    scratch += [
        pltpu.VMEM((CPW, B), jnp.int32),
        pltpu.VMEM((CPW, B), jnp.int32),
        pltpu.VMEM((NBUF, B, D_H), _f32),
        pltpu.VMEM((RPS // 4, D_H), _f32),
    ]
    if with_counts:
        scratch += [
            pltpu.VMEM((B, CNTW), _f32),
            pltpu.VMEM((RPS // 4, CNTW), _f32),
            pltpu.SemaphoreType.DMA,
        ]
    scratch += [pltpu.SemaphoreType.DMA] * NBUF

    return pl.kernel(body, mesh=_MESH, out_type=out_type,
                     scratch_types=scratch, compiler_params=_SC_PARAMS)


_seg_sum_counts = _make_seg_sum(True)
_seg_sum_plain = _make_seg_sum(False)


def _dot_t(a, w):
    return lax.dot_general(a, w, (((1,), (1,)), ((), ())),
                           preferred_element_type=_f32)


def _dense_in_body(x_ref, wl_ref, wr_ref, b_ref, xl_ref, sf_ref):
    x = x_ref[...]
    xl_ref[...] = _dot_t(x, wl_ref[...])
    sf_ref[...] = _dot_t(x, wr_ref[...]) + b_ref[...]


_dense_in = pl.pallas_call(
    _dense_in_body,
    out_shape=(jax.ShapeDtypeStruct((N, D_H), _f32),
               jax.ShapeDtypeStruct((N, D_H), _f32)),
)


def _mid_body(p_ref, c_ref, sf_ref, wl_ref, wr_ref, b_ref, hl_ref, sf2_ref):
    ssum = p_ref[0, :N, :] + p_ref[1, :N, :]
    cnt = c_ref[0, :N, 0:1] + c_ref[1, :N, 0:1]
    h = jnp.maximum(ssum / jnp.maximum(cnt, 1.0) + sf_ref[...], 0.0)
    hl_ref[...] = _dot_t(h, wl_ref[...])
    sf2_ref[...] = _dot_t(h, wr_ref[...]) + b_ref[...]


_mid = pl.pallas_call(
    _mid_body,
    out_shape=(jax.ShapeDtypeStruct((N, D_H), _f32),
               jax.ShapeDtypeStruct((N, D_H), _f32)),
)


def _final_body(p_ref, c_ref, sf_ref, wo_ref, bo_ref, out_ref):
    ssum = p_ref[0, :N, :] + p_ref[1, :N, :]
    cnt = c_ref[0, :N, 0:1] + c_ref[1, :N, 0:1]
    h = jnp.maximum(ssum / jnp.maximum(cnt, 1.0) + sf_ref[...], 0.0)
    out_ref[...] = _dot_t(h, wo_ref[...]) + bo_ref[...]


_final = pl.pallas_call(
    _final_body,
    out_shape=jax.ShapeDtypeStruct((N, 128), _f32),
)


def kernel(x, edge_index, W1l, b1l, W1r, W2l, b2l, W2r, Wout, bout):
    src3 = edge_index[0].reshape(NW, CPW, B)
    dst3 = edge_index[1].reshape(NW, CPW, B)

    xl, sf1 = _dense_in(x, W1l, W1r, b1l.reshape(1, D_H))
    psum1, pcnt = _seg_sum_counts(xl, src3, dst3)
    hl, sf2 = _mid(psum1, pcnt, sf1, W2l, W2r, b2l.reshape(1, D_H))
    psum2, = _seg_sum_plain(hl, src3, dst3)

    wo_pad = jnp.zeros((128, D_H), _f32).at[:2, :].set(Wout)
    bo_pad = jnp.zeros((1, 128), _f32).at[0, :2].set(bout)
    out_pad = _final(psum2, pcnt, sf2, wo_pad, bo_pad)
    return out_pad[:, :2]


# speedup vs baseline: 1.1588x; 1.0121x over previous
import functools

import jax
import jax.numpy as jnp
from jax import lax
from jax.experimental import pallas as pl
from jax.experimental.pallas import tpu as pltpu
from jax.experimental.pallas import tpu_sc as plsc

N = 10000
E = 640000
D_IN = 128
D_H = 64

NC = 2
NS = 16
NW = NC * NS
B = 80
CPW = E // (NW * B)
NPAD = 10240
RPS = NPAD // NS
CNTW = 16

_f32 = jnp.float32

_MESH = plsc.VectorSubcoreMesh(core_axis_name="c", subcore_axis_name="s")
_SC_PARAMS = pltpu.CompilerParams(use_tc_tiling_on_sc=False)


def _zero_rows(ref, rows, width):
    zero16 = jnp.zeros((16,), _f32)

    def zrow(r, _):
        for k in range(width // 16):
            ref[r, pl.ds(k * 16, 16)] = zero16
        return 0

    lax.fori_loop(0, rows, zrow, 0)


NBUF = 5
GROUPS = CPW // NBUF


EPW = E // NW


def _make_seg_sum(with_counts):
    def body(rows_hbm, edge_hbm, *rest):
        if with_counts:
            (out_sum, out_cnt, acc_sh, cnt_sh, src_v, dst_v, gath_v, zbuf_v,
             ones_v, zcnt_v, csem, *gsem) = rest
        else:
            out_sum, acc_sh, src_v, dst_v, gath_v, zbuf_v, *gsem = rest

        c = lax.axis_index("c")
        s = lax.axis_index("s")
        wid = s * NC + c

        _zero_rows(zbuf_v, RPS // 4, D_H)
        for q in range(4):
            pltpu.sync_copy(zbuf_v, acc_sh.at[pl.ds(s * RPS + q * (RPS // 4),
                                                    RPS // 4)])
        if with_counts:
            _zero_rows(zcnt_v, RPS // 4, CNTW)
            one16 = jnp.ones((16,), _f32)

            def orow(r, _):
                ones_v[r, pl.ds(0, CNTW)] = one16
                return 0

            lax.fori_loop(0, B, orow, 0)
            for q in range(4):
                pltpu.sync_copy(zcnt_v,
                                cnt_sh.at[pl.ds(s * RPS + q * (RPS // 4),
                                                RPS // 4)])

        pltpu.sync_copy(edge_hbm.at[0, pl.ds(wid * EPW, EPW)], src_v)
        pltpu.sync_copy(edge_hbm.at[1, pl.ds(wid * EPW, EPW)], dst_v)

        plsc.subcore_barrier()

        def gather(chunk, b):
            pltpu.async_copy(rows_hbm.at[src_v.at[pl.ds(chunk * B, B)]],
                             gath_v.at[b], gsem[b])

        def gwait(b):
            pltpu.make_async_copy(rows_hbm.at[src_v.at[pl.ds(0, B)]],
                                  gath_v.at[b], gsem[b]).wait()

        def scatter(chunk, b):
            pltpu.sync_copy(gath_v.at[b],
                            acc_sh.at[dst_v.at[pl.ds(chunk * B, B)]],
                            add=True)

        def cscatter(chunk):
            pltpu.async_copy(ones_v,
                             cnt_sh.at[dst_v.at[pl.ds(chunk * B, B)]], csem,
                             add=True)

        def cwait():
            pltpu.make_async_copy(ones_v, cnt_sh.at[dst_v.at[pl.ds(0, B)]],
                                  csem).wait()

        for b in range(NBUF):
            gather(b, b)

        def group(g, _):
            for b in range(NBUF):
                chunk = g * NBUF + b
                gwait(b)
                if with_counts:
                    @pl.when(chunk > 0)
                    def _():
                        cwait()
                    cscatter(chunk)
                scatter(chunk, b)
                gather(chunk + NBUF, b)
            return 0

        lax.fori_loop(0, GROUPS - 1, group, 0)

        for b in range(NBUF):
            chunk = (GROUPS - 1) * NBUF + b
            gwait(b)
            if with_counts:
                cwait()
                cscatter(chunk)
            scatter(chunk, b)
        if with_counts:
            cwait()

        plsc.subcore_barrier()

        for q in range(4):
            base = s * RPS + q * (RPS // 4)
            pltpu.sync_copy(acc_sh.at[pl.ds(base, RPS // 4)], zbuf_v)
            pltpu.sync_copy(zbuf_v, out_sum.at[c, pl.ds(base, RPS // 4)])
        if with_counts:
            for q in range(4):
                base = s * RPS + q * (RPS // 4)
                pltpu.sync_copy(cnt_sh.at[pl.ds(base, RPS // 4)], zcnt_v)
                pltpu.sync_copy(zcnt_v, out_cnt.at[c, pl.ds(base, RPS // 4)])

    out_type = [pltpu.HBM((NC, NPAD, D_H), _f32)]
    scratch = [
        pltpu.VMEM_SHARED((NPAD, D_H), _f32),
    ]
    if with_counts:
        out_type.append(pltpu.HBM((NC, NPAD, CNTW), _f32))
        scratch.append(pltpu.VMEM_SHARED((NPAD, CNTW), _f32))
    scratch += [
        pltpu.VMEM((EPW,), jnp.int32),
        pltpu.VMEM((EPW,), jnp.int32),
        pltpu.VMEM((NBUF, B, D_H), _f32),
        pltpu.VMEM((RPS // 4, D_H), _f32),
    ]
    if with_counts:
        scratch += [
            pltpu.VMEM((B, CNTW), _f32),
            pltpu.VMEM((RPS // 4, CNTW), _f32),
            pltpu.SemaphoreType.DMA,
        ]
    scratch += [pltpu.SemaphoreType.DMA] * NBUF

    return pl.kernel(body, mesh=_MESH, out_type=out_type,
                     scratch_types=scratch, compiler_params=_SC_PARAMS)


_seg_sum_counts = _make_seg_sum(True)
_seg_sum_plain = _make_seg_sum(False)


def _dot_t(a, w):
    return lax.dot_general(a, w, (((1,), (1,)), ((), ())),
                           preferred_element_type=_f32)


def _dense_in_body(x_ref, wl_ref, wr_ref, b_ref, xl_ref, sf_ref):
    x = x_ref[...]
    xl_ref[...] = _dot_t(x, wl_ref[...])
    sf_ref[...] = _dot_t(x, wr_ref[...]) + b_ref[...]


_dense_in = pl.pallas_call(
    _dense_in_body,
    out_shape=(jax.ShapeDtypeStruct((N, D_H), _f32),
               jax.ShapeDtypeStruct((N, D_H), _f32)),
)


def _mid_body(p_ref, c_ref, sf_ref, wl_ref, wr_ref, b_ref, hl_ref, sf2_ref):
    ssum = p_ref[0, :N, :] + p_ref[1, :N, :]
    cnt = c_ref[0, :N, 0:1] + c_ref[1, :N, 0:1]
    h = jnp.maximum(ssum / jnp.maximum(cnt, 1.0) + sf_ref[...], 0.0)
    hl_ref[...] = _dot_t(h, wl_ref[...])
    sf2_ref[...] = _dot_t(h, wr_ref[...]) + b_ref[...]


_mid = pl.pallas_call(
    _mid_body,
    out_shape=(jax.ShapeDtypeStruct((N, D_H), _f32),
               jax.ShapeDtypeStruct((N, D_H), _f32)),
)


def _final_body(p_ref, c_ref, sf_ref, wo_ref, bo_ref, out_ref):
    ssum = p_ref[0, :N, :] + p_ref[1, :N, :]
    cnt = c_ref[0, :N, 0:1] + c_ref[1, :N, 0:1]
    h = jnp.maximum(ssum / jnp.maximum(cnt, 1.0) + sf_ref[...], 0.0)
    out_ref[...] = _dot_t(h, wo_ref[...]) + bo_ref[...]


_final = pl.pallas_call(
    _final_body,
    out_shape=jax.ShapeDtypeStruct((N, 2), _f32),
)


def kernel(x, edge_index, W1l, b1l, W1r, W2l, b2l, W2r, Wout, bout):
    xl, sf1 = _dense_in(x, W1l, W1r, b1l.reshape(1, D_H))
    psum1, pcnt = _seg_sum_counts(xl, edge_index)
    hl, sf2 = _mid(psum1, pcnt, sf1, W2l, W2r, b2l.reshape(1, D_H))
    psum2, = _seg_sum_plain(hl, edge_index)
    return _final(psum2, pcnt, sf2, Wout, bout.reshape(1, 2))
